# Initial kernel scaffold; baseline (speedup 1.0000x reference)
import jax
import jax.numpy as jnp
from jax.experimental import pallas as pl


def kernel(coord, numbers, charge, mult, emb_table, Wc, sae_tensor):
    raise NotImplementedError("write your pallas kernel here")


# speedup vs baseline: 10.0174x; 10.0174x over previous
import jax
import jax.numpy as jnp
from jax import lax
from jax.experimental import pallas as pl

_H2EV = 27.211386245988
_BB = 128
_Z = 128


def _energy_body(c3s_ref, num_ref, embT_ref, wc_ref, sae_ref, sae0_ref, out_ref):
    Bb, N = num_ref.shape
    D, Z = embT_ref.shape
    cx = c3s_ref[0]
    cy = c3s_ref[1]
    cz = c3s_ref[2]
    w0 = wc_ref[0, :].reshape(1, 1, D)
    w1 = wc_ref[1, :].reshape(1, 1, D)
    w2 = wc_ref[2, :].reshape(1, 1, D)
    cf3 = jnp.tanh(cx[:, :, None] * w0 + cy[:, :, None] * w1 + cz[:, :, None] * w2)
    cf2 = cf3.reshape(Bb * N, D)
    g2 = jnp.dot(cf2, embT_ref[...], preferred_element_type=jnp.float32)
    g3 = g2.reshape(Bb, N, Z)
    num = num_ref[...]
    col = num - 1
    zidx = lax.broadcasted_iota(jnp.int32, (Bb, N, Z), 2)
    eq = zidx == col[:, :, None]
    val = g3 + sae_ref[...].reshape(1, 1, Z)
    per_atom = jnp.sum(jnp.where(eq, val, 0.0), axis=-1)
    pad_cnt = jnp.sum(jnp.where(num <= 0, 1.0, 0.0), axis=-1)
    energy = (jnp.sum(per_atom, axis=-1) + sae0_ref[0, 0] * pad_cnt) * _H2EV
    out_ref[0, 0, :] = energy


def kernel(coord, numbers, charge, mult, emb_table, Wc, sae_tensor):
    B, N, _ = coord.shape
    D = emb_table.shape[1]
    numbers = numbers.astype(jnp.int32)
    c3s = jnp.transpose(coord, (2, 0, 1))
    embT = jnp.transpose(emb_table[101:101 + _Z, :], (1, 0))
    sae_row = sae_tensor[101:101 + _Z].reshape(1, _Z)
    sae0 = sae_tensor[0:1].reshape(1, 1)
    nblk = B // _BB
    out = pl.pallas_call(
        _energy_body,
        grid=(nblk,),
        in_specs=[
            pl.BlockSpec((3, _BB, N), lambda i: (0, i, 0)),
            pl.BlockSpec((_BB, N), lambda i: (i, 0)),
            pl.BlockSpec((D, _Z), lambda i: (0, 0)),
            pl.BlockSpec((3, D), lambda i: (0, 0)),
            pl.BlockSpec((1, _Z), lambda i: (0, 0)),
            pl.BlockSpec((1, 1), lambda i: (0, 0)),
        ],
        out_specs=pl.BlockSpec((1, 1, _BB), lambda i: (i, 0, 0)),
        out_shape=jax.ShapeDtypeStruct((nblk, 1, _BB), jnp.float32),
    )(c3s, numbers, embT, Wc, sae_row, sae0)
    energy = out.reshape(B)
    return (energy, coord, numbers, charge, mult)


# speedup vs baseline: 12.6698x; 1.2648x over previous
import jax
import jax.numpy as jnp
from jax import lax
from jax.experimental import pallas as pl

_H2EV = 27.211386245988
_BB = 128
_Z = 128


def _energy_body(c3_ref, num_ref, embT_ref, wc_ref, sae_ref, out_ref):
    Bb, N = num_ref.shape
    D, Z = embT_ref.shape
    cf2 = jnp.tanh(jnp.dot(c3_ref[...], wc_ref[...],
                           preferred_element_type=jnp.float32))
    g2 = jnp.dot(cf2, embT_ref[...], preferred_element_type=jnp.float32)
    g3 = g2.reshape(Bb, N, Z)
    col = (num_ref[...] - 1) & (Z - 1)
    zidx = lax.broadcasted_iota(jnp.int32, (Bb, N, Z), 2)
    eq = zidx == col[:, :, None]
    val = g3 + sae_ref[...].reshape(1, 1, Z)
    per_atom = jnp.sum(jnp.where(eq, val, 0.0), axis=-1)
    out_ref[0, 0, :] = jnp.sum(per_atom, axis=-1) * _H2EV


def kernel(coord, numbers, charge, mult, emb_table, Wc, sae_tensor):
    B, N, _ = coord.shape
    D = emb_table.shape[1]
    numbers = numbers.astype(jnp.int32)
    c3 = coord.reshape(B * N, 3)
    embT = jnp.zeros((D, _Z), jnp.float32).at[:, :118].set(
        jnp.transpose(emb_table[101:219, :], (1, 0)))
    sae_row = jnp.zeros((1, _Z), jnp.float32).at[0, :118].set(
        sae_tensor[101:219]).at[0, _Z - 1].set(sae_tensor[0])
    nblk = B // _BB
    out = pl.pallas_call(
        _energy_body,
        grid=(nblk,),
        in_specs=[
            pl.BlockSpec((_BB * N, 3), lambda i: (i, 0)),
            pl.BlockSpec((_BB, N), lambda i: (i, 0)),
            pl.BlockSpec((D, _Z), lambda i: (0, 0)),
            pl.BlockSpec((3, D), lambda i: (0, 0)),
            pl.BlockSpec((1, _Z), lambda i: (0, 0)),
        ],
        out_specs=pl.BlockSpec((1, 1, _BB), lambda i: (i, 0, 0)),
        out_shape=jax.ShapeDtypeStruct((nblk, 1, _BB), jnp.float32),
    )(c3, numbers, embT, Wc, sae_row)
    energy = out.reshape(B)
    return (energy, coord, numbers, charge, mult)
